# Initial kernel scaffold; baseline (speedup 1.0000x reference)
import jax
import jax.numpy as jnp
from jax.experimental import pallas as pl


def kernel(indices, embeddings):
    raise NotImplementedError("write your pallas kernel here")


# speedup vs baseline: 2.7257x; 2.7257x over previous
import functools

import jax
import jax.numpy as jnp
from jax import lax
from jax.experimental import pallas as pl
from jax.experimental.pallas import tpu as pltpu
from jax.experimental.pallas import tpu_sc as plsc

NUM_EMBEDDINGS = 8192
EMBEDDING_DIM = 64
BATCH = 32
TOKENS = 1024

_NC = 2
_NS = 16
_NW = _NC * _NS
_B = BATCH * TOKENS
_BPW = _B // _NW
_CHUNK = 128
_NCHUNK = _BPW // _CHUNK


def _gather_body(table_hbm, idx_hbm, out_hbm, idx_v, rows_v, sem):
    wid = lax.axis_index("s") * _NC + lax.axis_index("c")
    base = wid * _BPW
    pltpu.sync_copy(idx_hbm.at[pl.ds(base, _BPW)], idx_v)
    copies = []
    for j in range(_NCHUNK):
        copies.append(
            pltpu.async_copy(
                table_hbm.at[idx_v.at[pl.ds(j * _CHUNK, _CHUNK)]],
                rows_v.at[pl.ds(j * _CHUNK, _CHUNK)],
                sem,
            )
        )
    for c in copies:
        c.wait()
    pltpu.sync_copy(rows_v, out_hbm.at[pl.ds(base, _BPW)])


_gather_call = pl.kernel(
    _gather_body,
    out_type=jax.ShapeDtypeStruct((_B, EMBEDDING_DIM), jnp.float32),
    mesh=plsc.VectorSubcoreMesh(core_axis_name="c", subcore_axis_name="s"),
    scratch_types=[
        pltpu.VMEM((_BPW,), jnp.int32),
        pltpu.VMEM((_BPW, EMBEDDING_DIM), jnp.float32),
        pltpu.SemaphoreType.DMA,
    ],
    compiler_params=pltpu.CompilerParams(use_tc_tiling_on_sc=False),
)


@jax.jit
def kernel(indices, embeddings):
    flat_idx = jnp.asarray(indices, jnp.int32).reshape(_B)
    out = _gather_call(embeddings, flat_idx)
    return out.reshape(BATCH, TOKENS, EMBEDDING_DIM)


# speedup vs baseline: 2.7389x; 1.0049x over previous
import functools

import jax
import jax.numpy as jnp
from jax import lax
from jax.experimental import pallas as pl
from jax.experimental.pallas import tpu as pltpu
from jax.experimental.pallas import tpu_sc as plsc

NUM_EMBEDDINGS = 8192
EMBEDDING_DIM = 64
BATCH = 32
TOKENS = 1024

_NC = 2
_NS = 16
_NW = _NC * _NS
_B = BATCH * TOKENS
_BPW = _B // _NW
_CHUNK = 128
_NCHUNK = _BPW // _CHUNK


def _gather_body(table_hbm, idx_hbm, out_hbm, idx_v, rows_v, sem):
    wid = lax.axis_index("s") * _NC + lax.axis_index("c")
    pltpu.sync_copy(idx_hbm.at[wid], idx_v)
    copies = []
    for j in range(_NCHUNK):
        copies.append(
            pltpu.async_copy(
                table_hbm.at[idx_v.at[pl.ds(j * _CHUNK, _CHUNK)]],
                rows_v.at[pl.ds(j * _CHUNK, _CHUNK)],
                sem,
            )
        )
    for c in copies:
        c.wait()
    pltpu.sync_copy(rows_v, out_hbm.at[wid])


_gather_call = pl.kernel(
    _gather_body,
    out_type=jax.ShapeDtypeStruct((BATCH, TOKENS, EMBEDDING_DIM), jnp.float32),
    mesh=plsc.VectorSubcoreMesh(core_axis_name="c", subcore_axis_name="s"),
    scratch_types=[
        pltpu.VMEM((TOKENS,), jnp.int32),
        pltpu.VMEM((TOKENS, EMBEDDING_DIM), jnp.float32),
        pltpu.SemaphoreType.DMA,
    ],
    compiler_params=pltpu.CompilerParams(use_tc_tiling_on_sc=False),
)


@jax.jit
def kernel(indices, embeddings):
    return _gather_call(embeddings, jnp.asarray(indices, jnp.int32))


# speedup vs baseline: 2.7846x; 1.0167x over previous
import jax
import jax.numpy as jnp
from jax import lax
from jax.experimental import pallas as pl
from jax.experimental.pallas import tpu as pltpu
from jax.experimental.pallas import tpu_sc as plsc

NUM_EMBEDDINGS = 8192
EMBEDDING_DIM = 64
BATCH = 32
TOKENS = 1024

_NC = 2
_NS = 16
_CHUNK = 128
_NCHUNK = TOKENS // _CHUNK


def _gather_body(table_hbm, idx_hbm, out_hbm, idx_v,
                 rows_a, rows_b, t_a, t_b,
                 gsem_a, gsem_b, ssem_a, ssem_b):
    wid = lax.axis_index("s") * _NC + lax.axis_index("c")
    pltpu.sync_copy(idx_hbm.at[wid], idx_v)

    rows = [rows_a, rows_b]
    tbuf = [t_a, t_b]
    gsem = [gsem_a, gsem_b]
    ssem = [ssem_a, ssem_b]

    def fire_gather(c):
        return pltpu.async_copy(
            table_hbm.at[idx_v.at[pl.ds(c * _CHUNK, _CHUNK)]],
            rows[c % 2],
            gsem[c % 2],
        )

    iota = lax.iota(jnp.int32, 16)
    gathers = [fire_gather(0)]
    stores = [None, None]
    for c in range(_NCHUNK):
        if c + 1 < _NCHUNK:
            gathers.append(fire_gather(c + 1))
        gathers[c].wait()
        if stores[c % 2] is not None:
            stores[c % 2].wait()
        G = rows[c % 2]
        T = tbuf[c % 2]

        @plsc.parallel_loop(0, EMBEDDING_DIM, step=1, unroll=2)
        def _transpose(f):
            fh = f // 8
            fl = f % 8
            fvec = lax.broadcast(f, (16,))
            for tl0 in range(_CHUNK // 16):
                v = plsc.load_gather(G, [iota + (tl0 * 16), fvec])
                T[fh, fl, pl.ds(tl0 * 16, 16)] = v

        stores[c % 2] = pltpu.async_copy(T, out_hbm.at[wid, :, c], ssem[c % 2])
    stores[0].wait()
    stores[1].wait()


_gather_call = pl.kernel(
    _gather_body,
    out_type=jax.ShapeDtypeStruct((BATCH, 8, _NCHUNK, 8, _CHUNK), jnp.float32),
    mesh=plsc.VectorSubcoreMesh(core_axis_name="c", subcore_axis_name="s"),
    scratch_types=[
        pltpu.VMEM((TOKENS,), jnp.int32),
        pltpu.VMEM((_CHUNK, EMBEDDING_DIM), jnp.float32),
        pltpu.VMEM((_CHUNK, EMBEDDING_DIM), jnp.float32),
        pltpu.VMEM((8, 8, _CHUNK), jnp.float32),
        pltpu.VMEM((8, 8, _CHUNK), jnp.float32),
        pltpu.SemaphoreType.DMA,
        pltpu.SemaphoreType.DMA,
        pltpu.SemaphoreType.DMA,
        pltpu.SemaphoreType.DMA,
    ],
    compiler_params=pltpu.CompilerParams(
        use_tc_tiling_on_sc=False, needs_layout_passes=False
    ),
)


@jax.jit
def kernel(indices, embeddings):
    out5 = _gather_call(embeddings, jnp.asarray(indices, jnp.int32))
    return out5.transpose(0, 2, 4, 1, 3).reshape(BATCH, TOKENS, EMBEDDING_DIM)
